# baseline (speedup 1.0000x reference)
import functools

import jax
import jax.numpy as jnp
from jax import lax
from jax.experimental import pallas as pl
from jax.experimental.pallas import tpu as pltpu
from jax.experimental.pallas import tpu_sc as plsc

B = 16384
D_DENSE = 13
F = 26
V = 100000
E = 16
TOTAL = B * F

NC = 2
NS = 16
NW = NC * NS
PER_W = TOTAL // NW
CHUNK = 128
CPW = PER_W // CHUNK
GC = 13
NG = CPW // GC
GROUP_ROWS = GC * CHUNK


BPW = B // NW
BCH = BPW // CHUNK
JT = 4
OUT_ROWS = JT * B * 8


def _sc_gather_body(cats_hbm, tables_hbm, out_hbm, cats_v, idx_v, dst_v,
                    rows_v, sem_g, sem_s):
    wid = lax.axis_index("s") * NC + lax.axis_index("c")
    b0 = wid * BPW
    lane = lax.iota(jnp.int32, 16)

    pltpu.sync_copy(cats_hbm.at[pl.ds(b0, BPW)], cats_v)

    def f_body(f, carry):
        jbase = lax.div(f, 8) * (B * 8) + lax.rem(f, 8)
        fvec = jnp.broadcast_to(f, (16,))
        for c in range(BCH):
            for l in range(CHUNK // 16):
                bl = c * CHUNK + l * 16 + lane
                cval = plsc.load_gather(cats_v, [bl, fvec])
                idx_v[c, pl.ds(l * 16, 16)] = cval
                dst_v[c, pl.ds(l * 16, 16)] = (b0 + bl) * 8 + jbase
        hs = [
            pltpu.async_copy(
                tables_hbm.at[f].at[idx_v.at[c]],
                rows_v.at[pl.ds(c * CHUNK, CHUNK)],
                sem_g,
            )
            for c in range(BCH)
        ]
        for h in hs:
            h.wait()
        ss = [
            pltpu.async_copy(
                rows_v.at[pl.ds(c * CHUNK, CHUNK)],
                out_hbm.at[dst_v.at[c]],
                sem_s,
            )
            for c in range(BCH)
        ]
        for s in ss:
            s.wait()
        return carry

    lax.fori_loop(0, F, f_body, 0)


@functools.cache
def _sc_gather():
    return pl.kernel(
        _sc_gather_body,
        out_type=jax.ShapeDtypeStruct((OUT_ROWS, E), jnp.float32),
        mesh=plsc.VectorSubcoreMesh(
            core_axis_name="c", subcore_axis_name="s",
            num_cores=NC, num_subcores=NS),
        scratch_types=[
            pltpu.VMEM((BPW, CHUNK), jnp.int32),
            pltpu.VMEM((BCH, CHUNK), jnp.int32),
            pltpu.VMEM((BCH, CHUNK), jnp.int32),
            pltpu.VMEM((BPW, E), jnp.float32),
            pltpu.SemaphoreType.DMA,
            pltpu.SemaphoreType.DMA,
        ],
        compiler_params=pltpu.CompilerParams(
            use_tc_tiling_on_sc=False, needs_layout_passes=False),
    )


BB = 1024


def _mlp_body(xd_ref, xe_ref, w1d_ref, w1e_ref, b1_ref, w2_ref, b2_ref,
              w3_ref, b3_ref, w4_ref, b4_ref, wld_ref, wle_ref, bl_ref,
              out_ref):
    f32 = jnp.float32
    hi = jax.lax.Precision.HIGHEST

    def dot(a, b):
        return jnp.dot(a, b, precision=hi, preferred_element_type=f32)

    xd = xd_ref[...]
    col = lax.broadcasted_iota(jnp.int32, (BB, 128), 1)
    h = dot(xd, w1d_ref[...]) + b1_ref[...]
    y_lin = dot(xd, wld_ref[...]) + bl_ref[...]
    for j in range(JT):
        xj = xe_ref[j]
        if j == JT - 1:
            xj = jnp.where(col < (F % 8) * E, xj, 0.0)
        h = h + dot(xj, w1e_ref[j * 128:(j + 1) * 128, :])
        y_lin = y_lin + dot(xj, wle_ref[j * 128:(j + 1) * 128, :])
    h = jnp.maximum(h, 0.0)
    h = jnp.maximum(dot(h, w2_ref[...]) + b2_ref[...], 0.0)
    h = jnp.maximum(dot(h, w3_ref[...]) + b3_ref[...], 0.0)
    y_deep = dot(h, w4_ref[...]) + b4_ref[...]
    out_ref[...] = jax.nn.sigmoid(y_lin + y_deep)


def _full(shape):
    return pl.BlockSpec(shape, lambda i: (0, 0))


def kernel(dense, cats, tables, W_lin, b_lin, W1, b1, W2, b2, W3, b3, W4, b4):
    cats_pad = jnp.pad(cats, ((0, 0), (0, CHUNK - F)))
    emb = _sc_gather()(cats_pad, tables)
    xe = emb.reshape(JT, B, 128)

    w1d, w1e = W1[:D_DENSE], W1[D_DENSE:]
    wld, wle = W_lin[:D_DENSE], W_lin[D_DENSE:]
    pad = ((0, JT * 128 - F * E), (0, 0))
    w1e = jnp.pad(w1e, pad)
    wle = jnp.pad(wle, pad)

    mlp = pl.pallas_call(
        _mlp_body,
        grid=(B // BB,),
        in_specs=[
            pl.BlockSpec((BB, D_DENSE), lambda i: (i, 0)),
            pl.BlockSpec((JT, BB, 128), lambda i: (0, i, 0)),
            _full((D_DENSE, 256)), _full((JT * 128, 256)), _full((1, 256)),
            _full((256, 128)), _full((1, 128)),
            _full((128, 64)), _full((1, 64)),
            _full((64, 1)), _full((1, 1)),
            _full((D_DENSE, 1)), _full((JT * 128, 1)), _full((1, 1)),
        ],
        out_specs=pl.BlockSpec((BB, 1), lambda i: (i, 0)),
        out_shape=jax.ShapeDtypeStruct((B, 1), jnp.float32),
    )
    return mlp(dense, xe,
               w1d, w1e, b1.reshape(1, -1),
               W2, b2.reshape(1, -1),
               W3, b3.reshape(1, -1),
               W4, b4.reshape(1, -1),
               wld, wle, b_lin.reshape(1, -1))


# speedup vs baseline: 1.1267x; 1.1267x over previous
import functools

import jax
import jax.numpy as jnp
from jax import lax
from jax.experimental import pallas as pl
from jax.experimental.pallas import tpu as pltpu
from jax.experimental.pallas import tpu_sc as plsc

B = 16384
D_DENSE = 13
F = 26
V = 100000
E = 16
TOTAL = B * F

NC = 2
NS = 16
NW = NC * NS
PER_W = TOTAL // NW
CHUNK = 128
CPW = PER_W // CHUNK
GC = 13
NG = CPW // GC
GROUP_ROWS = GC * CHUNK


BPW = B // NW
BCH = BPW // CHUNK
JT = 4
OUT_ROWS = JT * B * 8


def _sc_gather_body(cats_hbm, tables_hbm, out_hbm, cats_v, idx_v, dst_v,
                    rows_v, sem_g, sem_s):
    wid = lax.axis_index("s") * NC + lax.axis_index("c")
    b0 = wid * BPW
    lane = lax.iota(jnp.int32, 16)

    pltpu.sync_copy(cats_hbm.at[pl.ds(b0, BPW)], cats_v)

    def f_body(f, carry):
        jbase = lax.div(f, 8) * (B * 8) + lax.rem(f, 8)
        fvec = jnp.broadcast_to(f, (16,))
        for c in range(BCH):
            for l in range(CHUNK // 16):
                bl = c * CHUNK + l * 16 + lane
                cval = plsc.load_gather(cats_v, [bl, fvec])
                idx_v[c, pl.ds(l * 16, 16)] = cval
                dst_v[c, pl.ds(l * 16, 16)] = (b0 + bl) * 8 + jbase
        hs = [
            pltpu.async_copy(
                tables_hbm.at[f].at[idx_v.at[c]],
                rows_v.at[pl.ds(c * CHUNK, CHUNK)],
                sem_g,
            )
            for c in range(BCH)
        ]
        for h in hs:
            h.wait()
        ss = [
            pltpu.async_copy(
                rows_v.at[pl.ds(c * CHUNK, CHUNK)],
                out_hbm.at[dst_v.at[c]],
                sem_s,
            )
            for c in range(BCH)
        ]
        for s in ss:
            s.wait()
        return carry

    lax.fori_loop(0, F, f_body, 0)


@functools.cache
def _sc_gather():
    return pl.kernel(
        _sc_gather_body,
        out_type=jax.ShapeDtypeStruct((OUT_ROWS, E), jnp.float32),
        mesh=plsc.VectorSubcoreMesh(
            core_axis_name="c", subcore_axis_name="s",
            num_cores=NC, num_subcores=NS),
        scratch_types=[
            pltpu.VMEM((BPW, CHUNK), jnp.int32),
            pltpu.VMEM((BCH, CHUNK), jnp.int32),
            pltpu.VMEM((BCH, CHUNK), jnp.int32),
            pltpu.VMEM((BPW, E), jnp.float32),
            pltpu.SemaphoreType.DMA,
            pltpu.SemaphoreType.DMA,
        ],
        compiler_params=pltpu.CompilerParams(
            use_tc_tiling_on_sc=False, needs_layout_passes=False),
    )


BB = 1024


def _mlp_body(xd_ref, xe_ref, w1d_ref, w1e_ref, b1_ref, w2_ref, b2_ref,
              w3_ref, b3_ref, w4_ref, b4_ref, wld_ref, wle_ref, bl_ref,
              out_ref):
    f32 = jnp.float32

    def dot(a, b):
        return jnp.dot(a, b, preferred_element_type=f32)

    xd = xd_ref[...]
    col = lax.broadcasted_iota(jnp.int32, (BB, 128), 1)
    h = dot(xd, w1d_ref[...]) + b1_ref[...]
    y_lin = dot(xd, wld_ref[...]) + bl_ref[...]
    for j in range(JT):
        xj = xe_ref[j]
        if j == JT - 1:
            xj = jnp.where(col < (F % 8) * E, xj, 0.0)
        h = h + dot(xj, w1e_ref[j * 128:(j + 1) * 128, :])
        y_lin = y_lin + dot(xj, wle_ref[j * 128:(j + 1) * 128, :])
    h = jnp.maximum(h, 0.0)
    h = jnp.maximum(dot(h, w2_ref[...]) + b2_ref[...], 0.0)
    h = jnp.maximum(dot(h, w3_ref[...]) + b3_ref[...], 0.0)
    y_deep = dot(h, w4_ref[...]) + b4_ref[...]
    out_ref[...] = jax.nn.sigmoid(y_lin + y_deep)


def _full(shape):
    return pl.BlockSpec(shape, lambda i: (0, 0))


def kernel(dense, cats, tables, W_lin, b_lin, W1, b1, W2, b2, W3, b3, W4, b4):
    cats_pad = jnp.pad(cats, ((0, 0), (0, CHUNK - F)))
    emb = _sc_gather()(cats_pad, tables)
    xe = emb.reshape(JT, B, 128)

    w1d, w1e = W1[:D_DENSE], W1[D_DENSE:]
    wld, wle = W_lin[:D_DENSE], W_lin[D_DENSE:]
    pad = ((0, JT * 128 - F * E), (0, 0))
    w1e = jnp.pad(w1e, pad)
    wle = jnp.pad(wle, pad)

    mlp = pl.pallas_call(
        _mlp_body,
        grid=(B // BB,),
        in_specs=[
            pl.BlockSpec((BB, D_DENSE), lambda i: (i, 0)),
            pl.BlockSpec((JT, BB, 128), lambda i: (0, i, 0)),
            _full((D_DENSE, 256)), _full((JT * 128, 256)), _full((1, 256)),
            _full((256, 128)), _full((1, 128)),
            _full((128, 64)), _full((1, 64)),
            _full((64, 1)), _full((1, 1)),
            _full((D_DENSE, 1)), _full((JT * 128, 1)), _full((1, 1)),
        ],
        out_specs=pl.BlockSpec((BB, 1), lambda i: (i, 0)),
        out_shape=jax.ShapeDtypeStruct((B, 1), jnp.float32),
    )
    return mlp(dense, xe,
               w1d, w1e, b1.reshape(1, -1),
               W2, b2.reshape(1, -1),
               W3, b3.reshape(1, -1),
               W4, b4.reshape(1, -1),
               wld, wle, b_lin.reshape(1, -1))


# speedup vs baseline: 1.1293x; 1.0023x over previous
import functools

import jax
import jax.numpy as jnp
from jax import lax
from jax.experimental import pallas as pl
from jax.experimental.pallas import tpu as pltpu
from jax.experimental.pallas import tpu_sc as plsc

B = 16384
D_DENSE = 13
F = 26
V = 100000
E = 16
TOTAL = B * F

NC = 2
NS = 16
NW = NC * NS
PER_W = TOTAL // NW
CHUNK = 128
CPW = PER_W // CHUNK
GC = 13
NG = CPW // GC
GROUP_ROWS = GC * CHUNK


BPW = B // NW
BCH = BPW // CHUNK
JT = 4
OUT_ROWS = JT * B * 8


def _sc_gather_body(cats_hbm, tables_hbm, out_hbm, cats_v, idx_v, dst_v,
                    rows_v, sem_g, sem_s):
    wid = lax.axis_index("s") * NC + lax.axis_index("c")
    b0 = wid * BPW
    lane = lax.iota(jnp.int32, 16)

    pltpu.sync_copy(cats_hbm.at[pl.ds(b0, BPW)], cats_v)

    def f_body(f, carry):
        jbase = lax.div(f, 8) * (B * 8) + lax.rem(f, 8)
        fvec = jnp.broadcast_to(f, (16,))
        for c in range(BCH):
            for l in range(CHUNK // 16):
                bl = c * CHUNK + l * 16 + lane
                cval = plsc.load_gather(cats_v, [bl, fvec])
                idx_v[c, pl.ds(l * 16, 16)] = cval
                dst_v[c, pl.ds(l * 16, 16)] = (b0 + bl) * 8 + jbase
        hs = [
            pltpu.async_copy(
                tables_hbm.at[f].at[idx_v.at[c]],
                rows_v.at[pl.ds(c * CHUNK, CHUNK)],
                sem_g,
            )
            for c in range(BCH)
        ]
        for h in hs:
            h.wait()
        ss = [
            pltpu.async_copy(
                rows_v.at[pl.ds(c * CHUNK, CHUNK)],
                out_hbm.at[dst_v.at[c]],
                sem_s,
            )
            for c in range(BCH)
        ]
        for s in ss:
            s.wait()
        return carry

    lax.fori_loop(0, F, f_body, 0)


@functools.cache
def _sc_gather():
    return pl.kernel(
        _sc_gather_body,
        out_type=jax.ShapeDtypeStruct((OUT_ROWS, E), jnp.float32),
        mesh=plsc.VectorSubcoreMesh(
            core_axis_name="c", subcore_axis_name="s",
            num_cores=NC, num_subcores=NS),
        scratch_types=[
            pltpu.VMEM((BPW, CHUNK), jnp.int32),
            pltpu.VMEM((BCH, CHUNK), jnp.int32),
            pltpu.VMEM((BCH, CHUNK), jnp.int32),
            pltpu.VMEM((BPW, E), jnp.float32),
            pltpu.SemaphoreType.DMA,
            pltpu.SemaphoreType.DMA,
        ],
        compiler_params=pltpu.CompilerParams(
            use_tc_tiling_on_sc=False, needs_layout_passes=False),
    )


BB = 2048


def _mlp_body(xd_ref, xe_ref, w1d_ref, w1e_ref, b1_ref, w2_ref, b2_ref,
              w3_ref, b3_ref, w4_ref, b4_ref, wld_ref, wle_ref, bl_ref,
              out_ref):
    f32 = jnp.float32

    def dot(a, b):
        return jnp.dot(a, b, preferred_element_type=f32)

    xd = xd_ref[...]
    col = lax.broadcasted_iota(jnp.int32, (BB, 128), 1)
    h = dot(xd, w1d_ref[...]) + b1_ref[...]
    y_lin = dot(xd, wld_ref[...]) + bl_ref[...]
    for j in range(JT):
        xj = xe_ref[j]
        if j == JT - 1:
            xj = jnp.where(col < (F % 8) * E, xj, 0.0)
        h = h + dot(xj, w1e_ref[j * 128:(j + 1) * 128, :])
        y_lin = y_lin + dot(xj, wle_ref[j * 128:(j + 1) * 128, :])
    h = jnp.maximum(h, 0.0)
    h = jnp.maximum(dot(h, w2_ref[...]) + b2_ref[...], 0.0)
    h = jnp.maximum(dot(h, w3_ref[...]) + b3_ref[...], 0.0)
    y_deep = dot(h, w4_ref[...]) + b4_ref[...]
    out_ref[...] = jax.nn.sigmoid(y_lin + y_deep)


def _full(shape):
    return pl.BlockSpec(shape, lambda i: (0, 0))


def kernel(dense, cats, tables, W_lin, b_lin, W1, b1, W2, b2, W3, b3, W4, b4):
    cats_pad = jnp.pad(cats, ((0, 0), (0, CHUNK - F)))
    emb = _sc_gather()(cats_pad, tables)
    xe = emb.reshape(JT, B, 128)

    w1d, w1e = W1[:D_DENSE], W1[D_DENSE:]
    wld, wle = W_lin[:D_DENSE], W_lin[D_DENSE:]
    pad = ((0, JT * 128 - F * E), (0, 0))
    w1e = jnp.pad(w1e, pad)
    wle = jnp.pad(wle, pad)

    mlp = pl.pallas_call(
        _mlp_body,
        grid=(B // BB,),
        in_specs=[
            pl.BlockSpec((BB, D_DENSE), lambda i: (i, 0)),
            pl.BlockSpec((JT, BB, 128), lambda i: (0, i, 0)),
            _full((D_DENSE, 256)), _full((JT * 128, 256)), _full((1, 256)),
            _full((256, 128)), _full((1, 128)),
            _full((128, 64)), _full((1, 64)),
            _full((64, 1)), _full((1, 1)),
            _full((D_DENSE, 1)), _full((JT * 128, 1)), _full((1, 1)),
        ],
        out_specs=pl.BlockSpec((BB, 1), lambda i: (i, 0)),
        out_shape=jax.ShapeDtypeStruct((B, 1), jnp.float32),
    )
    return mlp(dense, xe,
               w1d, w1e, b1.reshape(1, -1),
               W2, b2.reshape(1, -1),
               W3, b3.reshape(1, -1),
               W4, b4.reshape(1, -1),
               wld, wle, b_lin.reshape(1, -1))
